# baseline (speedup 1.0000x reference)
import jax
import jax.numpy as jnp
from jax.experimental import pallas as pl


def _dequant_kernel(s_ref, c_ref, o_ref):
    s = s_ref[0, 0].astype(jnp.bfloat16)
    c = c_ref[0].astype(jnp.bfloat16)
    o_ref[0, 0] = jax.lax.dot_general(
        c, s, (((0,), (1,)), ((), ())),
        preferred_element_type=jnp.float32)


def kernel(sample, codebook):
    n, m, h, w, k = sample.shape
    d = codebook.shape[-1]
    hw = h * w
    s = sample.reshape(n, m, hw, k)
    out = pl.pallas_call(
        _dequant_kernel,
        grid=(n, m),
        in_specs=[
            pl.BlockSpec((1, 1, hw, k), lambda i, j: (i, j, 0, 0)),
            pl.BlockSpec((1, k, d), lambda i, j: (j, 0, 0)),
        ],
        out_specs=pl.BlockSpec((1, 1, d, hw), lambda i, j: (i, j, 0, 0)),
        out_shape=jax.ShapeDtypeStruct((n, m, d, hw), jnp.float32),
    )(s, codebook)
    return out.reshape(n, m * d, h, w)


# speedup vs baseline: 1.0478x; 1.0478x over previous
import jax
import jax.numpy as jnp
from jax.experimental import pallas as pl


def _dequant_kernel(s_ref, c_ref, o_ref):
    s = s_ref[0, 0].astype(jnp.bfloat16)
    c = c_ref[0].astype(jnp.bfloat16)
    o_ref[0, 0] = jax.lax.dot_general(
        c, s, (((0,), (1,)), ((), ())),
        preferred_element_type=jnp.float32)


def kernel(sample, codebook):
    n, m, h, w, k = sample.shape
    d = codebook.shape[-1]
    hw = h * w
    s = sample.reshape(n, m, hw, k)
    out = pl.pallas_call(
        _dequant_kernel,
        grid=(m, n),
        in_specs=[
            pl.BlockSpec((1, 1, hw, k), lambda j, i: (i, j, 0, 0)),
            pl.BlockSpec((1, k, d), lambda j, i: (j, 0, 0)),
        ],
        out_specs=pl.BlockSpec((1, 1, d, hw), lambda j, i: (i, j, 0, 0)),
        out_shape=jax.ShapeDtypeStruct((n, m, d, hw), jnp.float32),
    )(s, codebook)
    return out.reshape(n, m * d, h, w)


# speedup vs baseline: 1.2240x; 1.1682x over previous
import jax
import jax.numpy as jnp
from jax.experimental import pallas as pl


_NB = 4


def _dequant_kernel(s_ref, c_ref, o_ref):
    c = c_ref[0].astype(jnp.bfloat16)
    for b in range(_NB):
        s = s_ref[b, 0].astype(jnp.bfloat16)
        o_ref[b, 0] = jax.lax.dot_general(
            c, s, (((0,), (1,)), ((), ())),
            preferred_element_type=jnp.float32)


def kernel(sample, codebook):
    n, m, h, w, k = sample.shape
    d = codebook.shape[-1]
    hw = h * w
    s = sample.reshape(n, m, hw, k)
    out = pl.pallas_call(
        _dequant_kernel,
        grid=(m, n // _NB),
        in_specs=[
            pl.BlockSpec((_NB, 1, hw, k), lambda j, i: (i, j, 0, 0)),
            pl.BlockSpec((1, k, d), lambda j, i: (j, 0, 0)),
        ],
        out_specs=pl.BlockSpec((_NB, 1, d, hw), lambda j, i: (i, j, 0, 0)),
        out_shape=jax.ShapeDtypeStruct((n, m, d, hw), jnp.float32),
    )(s, codebook)
    return out.reshape(n, m * d, h, w)
